# baseline (device time: 12786 ns/iter reference)
import jax
import jax.numpy as jnp
from jax import lax
from jax.experimental import pallas as pl
from jax.experimental.pallas import tpu as pltpu

N_DEV = 4


def kernel(x, router_W, route_idx, expert_W, shared_W):
    n_tok, d_model = x.shape
    n_local_exp, _, d_ff = expert_W.shape
    n_exp = router_W.shape[1]
    d_cat = n_local_exp * d_model

    rwT = router_W.T

    def body(x_ref, rwT_ref, idx_ref, ew_hbm, sw_hbm, out_ref,
             ewf_ref, ew16_ref, swf_ref, comm_ref,
             ew_dma_sem, sw_dma_sem, send_sems, recv_sems):
        my_pos = lax.axis_index("i")

        ew_dma = pltpu.make_async_copy(ew_hbm, ewf_ref, ew_dma_sem)
        ew_dma.start()
        sw_dma = pltpu.make_async_copy(sw_hbm, swf_ref, sw_dma_sem)
        sw_dma.start()

        barrier_sem = pltpu.get_barrier_semaphore()
        for k in range(1, N_DEV):
            pl.semaphore_signal(
                barrier_sem, inc=1,
                device_id=(lax.rem(my_pos + k, N_DEV),),
                device_id_type=pl.DeviceIdType.MESH,
            )
        pl.semaphore_wait(barrier_sem, N_DEV - 1)

        ew_dma.wait()
        ew16_ref[...] = ewf_ref[...].reshape(d_cat, d_ff).astype(jnp.bfloat16)

        sends = []
        for k in range(1, N_DEV):
            s = pltpu.make_async_remote_copy(
                src_ref=ew16_ref,
                dst_ref=comm_ref.at[N_DEV - k],
                send_sem=send_sems.at[k - 1],
                recv_sem=recv_sems.at[N_DEV - k],
                device_id=(lax.rem(my_pos + k, N_DEV),),
                device_id_type=pl.DeviceIdType.MESH,
            )
            s.start()
            sends.append(s)

        def recv_for(slot):
            return pltpu.make_async_remote_copy(
                src_ref=ew16_ref,
                dst_ref=comm_ref.at[slot],
                send_sem=send_sems.at[N_DEV - 1],
                recv_sem=recv_sems.at[slot],
                device_id=(my_pos,),
                device_id_type=pl.DeviceIdType.MESH,
            )

        xv = x_ref[...]
        eid = idx_ref[...]
        scores = lax.dot_general(
            xv, rwT_ref[...], (((1,), (1,)), ((), ())),
            preferred_element_type=jnp.float32)
        m = jnp.max(scores, axis=-1, keepdims=True)
        p = jnp.exp(scores - m)
        denom = jnp.sum(p, axis=-1, keepdims=True)
        onehot = lax.broadcasted_iota(jnp.int32, (n_tok, n_exp), 1) == eid
        gate = jnp.sum(jnp.where(onehot, p, 0.0), axis=-1, keepdims=True) / denom

        def scaled_x_for(origin):
            parts = []
            for j in range(n_local_exp):
                e = origin * n_local_exp + j
                w = jnp.where(eid == e, gate, 0.0)
                parts.append((xv * w).astype(jnp.bfloat16))
            return jnp.concatenate(parts, axis=1)

        xm = [scaled_x_for(lax.rem(my_pos + s, N_DEV)) for s in range(N_DEV)]

        x16 = xv.astype(jnp.bfloat16)
        sw_dma.wait()
        acc = jnp.dot(x16, swf_ref[...].astype(jnp.bfloat16),
                      preferred_element_type=jnp.float32)
        acc = acc + jnp.dot(xm[0], ew16_ref[...],
                            preferred_element_type=jnp.float32)

        for slot in (1, 3, 2):
            recv_for(slot).wait_recv()
            acc = acc + jnp.dot(xm[slot], comm_ref[slot],
                                preferred_element_type=jnp.float32)

        for s in sends:
            s.wait_send()

        out_ref[...] = acc.astype(jnp.bfloat16)

    return pl.pallas_call(
        body,
        out_shape=jax.ShapeDtypeStruct((n_tok, d_ff), jnp.bfloat16),
        in_specs=[
            pl.BlockSpec(memory_space=pltpu.VMEM),
            pl.BlockSpec(memory_space=pltpu.VMEM),
            pl.BlockSpec(memory_space=pltpu.VMEM),
            pl.BlockSpec(memory_space=pl.ANY),
            pl.BlockSpec(memory_space=pl.ANY),
        ],
        out_specs=pl.BlockSpec(memory_space=pltpu.VMEM),
        scratch_shapes=[
            pltpu.VMEM((n_local_exp, d_model, d_ff), jnp.float32),
            pltpu.VMEM((d_cat, d_ff), jnp.bfloat16),
            pltpu.VMEM((d_model, d_ff), jnp.float32),
            pltpu.VMEM((N_DEV, d_cat, d_ff), jnp.bfloat16),
            pltpu.SemaphoreType.DMA,
            pltpu.SemaphoreType.DMA,
            pltpu.SemaphoreType.DMA((N_DEV,)),
            pltpu.SemaphoreType.DMA((N_DEV,)),
        ],
        compiler_params=pltpu.CompilerParams(collective_id=0),
    )(x, rwT, route_idx, expert_W, shared_W)


# device time: 12540 ns/iter; 1.0196x vs baseline; 1.0196x over previous
import jax
import jax.numpy as jnp
from jax import lax
from jax.experimental import pallas as pl
from jax.experimental.pallas import tpu as pltpu

N_DEV = 4


def kernel(x, router_W, route_idx, expert_W, shared_W):
    n_tok, d_model = x.shape
    n_local_exp, _, d_ff = expert_W.shape
    n_exp = router_W.shape[1]
    d_cat = n_local_exp * d_model

    rwT = router_W.T

    def body(x_ref, rwT_ref, idx_ref, ew_ref, sw_ref, out_ref,
             ew16_ref, comm_ref, send_sems, recv_sems):
        my_pos = lax.axis_index("i")

        barrier_sem = pltpu.get_barrier_semaphore()
        for k in range(1, N_DEV):
            pl.semaphore_signal(
                barrier_sem, inc=1,
                device_id=(lax.rem(my_pos + k, N_DEV),),
                device_id_type=pl.DeviceIdType.MESH,
            )
        ew16_ref[...] = ew_ref[...].reshape(d_cat, d_ff).astype(jnp.bfloat16)
        pl.semaphore_wait(barrier_sem, N_DEV - 1)

        sends = []
        for k in range(1, N_DEV):
            s = pltpu.make_async_remote_copy(
                src_ref=ew16_ref,
                dst_ref=comm_ref.at[N_DEV - k],
                send_sem=send_sems.at[k - 1],
                recv_sem=recv_sems.at[N_DEV - k],
                device_id=(lax.rem(my_pos + k, N_DEV),),
                device_id_type=pl.DeviceIdType.MESH,
            )
            s.start()
            sends.append(s)

        def recv_for(slot):
            return pltpu.make_async_remote_copy(
                src_ref=ew16_ref,
                dst_ref=comm_ref.at[slot],
                send_sem=send_sems.at[N_DEV - 1],
                recv_sem=recv_sems.at[slot],
                device_id=(my_pos,),
                device_id_type=pl.DeviceIdType.MESH,
            )

        xv = x_ref[...]
        eid = idx_ref[...]
        scores = lax.dot_general(
            xv, rwT_ref[...], (((1,), (1,)), ((), ())),
            preferred_element_type=jnp.float32)
        m = jnp.max(scores, axis=-1, keepdims=True)
        p = jnp.exp(scores - m)
        denom = jnp.sum(p, axis=-1, keepdims=True)
        onehot = lax.broadcasted_iota(jnp.int32, (n_tok, n_exp), 1) == eid
        gate = jnp.sum(jnp.where(onehot, p, 0.0), axis=-1, keepdims=True) / denom

        def scaled_x_for(origin):
            parts = []
            for j in range(n_local_exp):
                e = origin * n_local_exp + j
                w = jnp.where(eid == e, gate, 0.0)
                parts.append((xv * w).astype(jnp.bfloat16))
            return jnp.concatenate(parts, axis=1)

        xm = [scaled_x_for(lax.rem(my_pos + s, N_DEV)) for s in range(N_DEV)]

        x16 = xv.astype(jnp.bfloat16)
        acc = jnp.dot(x16, sw_ref[...].astype(jnp.bfloat16),
                      preferred_element_type=jnp.float32)
        acc = acc + jnp.dot(xm[0], ew16_ref[...],
                            preferred_element_type=jnp.float32)

        for slot in (1, 3, 2):
            recv_for(slot).wait_recv()
            acc = acc + jnp.dot(xm[slot], comm_ref[slot],
                                preferred_element_type=jnp.float32)

        for s in sends:
            s.wait_send()

        out_ref[...] = acc

    return pl.pallas_call(
        body,
        out_shape=jax.ShapeDtypeStruct((n_tok, d_ff), jnp.float32),
        in_specs=[pl.BlockSpec(memory_space=pltpu.VMEM)] * 5,
        out_specs=pl.BlockSpec(memory_space=pltpu.VMEM),
        scratch_shapes=[
            pltpu.VMEM((d_cat, d_ff), jnp.bfloat16),
            pltpu.VMEM((N_DEV, d_cat, d_ff), jnp.bfloat16),
            pltpu.SemaphoreType.DMA((N_DEV,)),
            pltpu.SemaphoreType.DMA((N_DEV,)),
        ],
        compiler_params=pltpu.CompilerParams(collective_id=0),
    )(x, rwT, route_idx, expert_W, shared_W)
